# baseline (speedup 1.0000x reference)
import jax
import jax.numpy as jnp
from jax import lax
from jax.experimental import pallas as pl
from jax.experimental.pallas import tpu as pltpu
from jax.experimental.pallas import tpu_sc as plsc

NC = 2
NS = 16
NW = NC * NS

WDIM = 64
PDIM = 32
B = 4096
L = 200
PV = 1000
PVP = 1008

LANES = 128
BT = B // LANES
SUBS = 8
UB = B // SUBS
JROWS = UB // LANES
HB = JROWS // 2
UNITS = L * SUBS
PER_W = UNITS // NW


@jax.jit
def _embed(words_t3, pos_t3, word_table, pos_table2):
  mesh = plsc.VectorSubcoreMesh(core_axis_name="c", subcore_axis_name="s")

  def body(words_hbm, pos_hbm, wtab_hbm, ptab_hbm, out_w_hbm, out_p_hbm,
           idx_w, idx_p, rows_w, ptab_t, tw, tp,
           sem_i, sem_g, sem_ww, sem_wp):
    wid = lax.axis_index("s") * NC + lax.axis_index("c")
    iota16 = lax.iota(jnp.int32, 16)
    g16 = [g * 16 + iota16 for g in range(8)]
    btl_splat = [jnp.full((16,), btl, jnp.int32) for btl in range(HB)]

    stage = rows_w.at[0, pl.ds(0, PV // 2)]
    pltpu.sync_copy(ptab_hbm, stage)

    @plsc.parallel_loop(0, PVP // 16)
    def build(ig):
      i_vec = ig * 16 + iota16
      i_clamped = jnp.minimum(i_vec, PV - 1)
      r_vec = i_clamped >> 1
      c_base = (i_clamped & 1) << 5
      for d in range(PDIM):
        v = plsc.load_gather(stage, [r_vec, c_base + d])
        ptab_t[d, pl.ds(ig * 16, 16)] = v

    def unit_lsub(k):
      u = wid * PER_W + k
      return u // SUBS, u % SUBS

    def idx_load(k, slot):
      l, sub = unit_lsub(k)
      pltpu.async_copy(words_hbm.at[l, pl.ds(sub * JROWS, JROWS)],
                       idx_w.at[slot], sem_i)
      pltpu.async_copy(pos_hbm.at[l, pl.ds(sub * JROWS, JROWS)],
                       idx_p.at[slot], sem_i)

    def idx_wait(slot):
      pltpu.make_async_copy(words_hbm.at[0, pl.ds(0, JROWS)],
                            idx_w.at[slot], sem_i).wait()
      pltpu.make_async_copy(pos_hbm.at[0, pl.ds(0, JROWS)],
                            idx_p.at[slot], sem_i).wait()

    def fire_gathers(slot):
      for j in range(JROWS):
        pltpu.async_copy(wtab_hbm.at[idx_w.at[slot, j]],
                         rows_w.at[slot, pl.ds(j * LANES, LANES)], sem_g)

    def drain_gathers():
      for j in range(JROWS):
        pltpu.make_async_copy(
            wtab_hbm.at[idx_w.at[0, 0]],
            rows_w.at[0, pl.ds(0, LANES)], sem_g).wait()

    def wb_wait():
      pltpu.make_async_copy(
          tw, out_w_hbm.at[0, :, pl.ds(0, HB)], sem_ww).wait()
      pltpu.make_async_copy(
          tp, out_p_hbm.at[0, :, pl.ds(0, HB)], sem_wp).wait()

    def transpose(rows, dim, base):
      @plsc.parallel_loop(0, dim, unroll=4)
      def t_body(c0):
        c_vec = (c0 + iota16) & (dim - 1)
        dt_vec = c_vec >> 3
        dr_vec = c_vec & 7
        for bl in range(HB):
          for g in range(8):
            r_vec = (base + bl) * LANES + g16[g]
            v = plsc.load_gather(rows, [r_vec, c_vec])
            plsc.store_scatter(tw, [dt_vec, btl_splat[bl], dr_vec,
                                    g16[g]], v)

    def pos_fill(slot, base):
      idxv = [[idx_p[slot, base + bl, pl.ds(g * 16, 16)]
               for g in range(8)] for bl in range(HB)]

      @plsc.parallel_loop(0, PDIM, unroll=4)
      def p_body(d):
        d_splat = jnp.full((16,), d, jnp.int32)
        dt = d >> 3
        dr = d & 7
        for bl in range(HB):
          for g in range(8):
            v = plsc.load_gather(ptab_t, [d_splat, idxv[bl][g]])
            tp[dt, bl, dr, pl.ds(g * 16, 16)] = v

    idx_load(0, 0)
    idx_wait(0)
    fire_gathers(0)
    idx_load(1, 1)

    def unit(k, carry):
      s = k & 1
      l, sub = unit_lsub(k)
      drain_gathers()

      @pl.when(k + 1 < PER_W)
      def _():
        idx_wait(1 - s)
        fire_gathers(1 - s)

      rw = rows_w.at[s]
      @pl.when(k > 0)
      def _():
        wb_wait()
      transpose(rw, WDIM, 0)
      pos_fill(s, 0)
      pltpu.async_copy(tw, out_w_hbm.at[l, :, pl.ds(sub * JROWS, HB)],
                       sem_ww)
      pltpu.async_copy(tp, out_p_hbm.at[l, :, pl.ds(sub * JROWS, HB)],
                       sem_wp)
      wb_wait()
      transpose(rw, WDIM, HB)
      pos_fill(s, HB)
      pltpu.async_copy(tw, out_w_hbm.at[l, :, pl.ds(sub * JROWS + HB, HB)],
                       sem_ww)
      pltpu.async_copy(tp, out_p_hbm.at[l, :, pl.ds(sub * JROWS + HB, HB)],
                       sem_wp)
      @pl.when(k + 2 < PER_W)
      def _():
        idx_load(k + 2, s)
      return carry

    lax.fori_loop(0, PER_W, unit, 0)
    wb_wait()

  run = pl.kernel(
      body,
      out_type=(
          jax.ShapeDtypeStruct((L, WDIM // 8, BT, 8, LANES), jnp.float32),
          jax.ShapeDtypeStruct((L, PDIM // 8, BT, 8, LANES), jnp.float32),
      ),
      mesh=mesh,
      compiler_params=pltpu.CompilerParams(use_tc_tiling_on_sc=False,
                                           needs_layout_passes=False),
      scratch_types=[
          pltpu.VMEM((2, JROWS, LANES), jnp.int32),
          pltpu.VMEM((2, JROWS, LANES), jnp.int32),
          pltpu.VMEM((2, UB, WDIM), jnp.float32),
          pltpu.VMEM((PDIM, PVP), jnp.float32),
          pltpu.VMEM((WDIM // 8, HB, 8, LANES), jnp.float32),
          pltpu.VMEM((PDIM // 8, HB, 8, LANES), jnp.float32),
          pltpu.SemaphoreType.DMA,
          pltpu.SemaphoreType.DMA,
          pltpu.SemaphoreType.DMA,
          pltpu.SemaphoreType.DMA,
      ],
  )
  return run(words_t3, pos_t3, word_table, pos_table2)


def kernel(words, pos, word_table, pos_table):
  words_t3 = words.astype(jnp.int32).T.reshape(L, BT, LANES)
  pos_t3 = pos.astype(jnp.int32).T.reshape(L, BT, LANES)
  pos_table2 = pos_table.reshape(PV // 2, 2 * PDIM)
  out_w5, out_p5 = _embed(words_t3, pos_t3, word_table, pos_table2)
  out_w = out_w5.transpose(2, 4, 0, 1, 3).reshape(B, L, WDIM)
  out_p = out_p5.transpose(2, 4, 0, 1, 3).reshape(B, L, PDIM)
  return (out_w, out_p)
